# baseline (speedup 1.0000x reference)
import functools

import jax
import jax.numpy as jnp
import numpy as np
from jax import lax
from jax.experimental import pallas as pl
from jax.experimental.pallas import tpu as pltpu
from jax.experimental.pallas import tpu_sc as plsc

MAX_PAIRS = 500000
NUM_CORES = 2
NUM_SUBCORES = 16
NUM_WORKERS = NUM_CORES * NUM_SUBCORES
LANES = 16
NCHUNK = 8

PAIRS_PER_WORKER = -(-MAX_PAIRS // (NUM_WORKERS * LANES)) * LANES
PADDED_PAIRS = PAIRS_PER_WORKER * NUM_WORKERS
VECS_PER_WORKER = PAIRS_PER_WORKER // LANES

_IDX_CACHE = {}

_U32 = np.uint64(0xFFFFFFFF)


def _threefry2x32(k1, k2, x0, x1):
    k1 = np.uint64(k1)
    k2 = np.uint64(k2)
    x0 = np.asarray(x0, np.uint64)
    x1 = np.asarray(x1, np.uint64)
    ks = [k1 & _U32, k2 & _U32, (k1 ^ k2 ^ np.uint64(0x1BD11BDA)) & _U32]
    rot = ([13, 15, 26, 6], [17, 29, 16, 24])
    x0 = (x0 + ks[0]) & _U32
    x1 = (x1 + ks[1]) & _U32
    for i in range(5):
        for r in rot[i % 2]:
            x0 = (x0 + x1) & _U32
            x1 = ((x1 << np.uint64(r)) | (x1 >> np.uint64(32 - r))) & _U32
            x1 = x0 ^ x1
        x0 = (x0 + ks[(i + 1) % 3]) & _U32
        x1 = (x1 + ks[(i + 2) % 3] + np.uint64(i + 1)) & _U32
    return x0.astype(np.uint32), x1.astype(np.uint32)


def _randint_np(seed, shape, minval, maxval):
    k1 = np.uint32((seed >> 32) & 0xFFFFFFFF)
    k2 = np.uint32(seed & 0xFFFFFFFF)
    size = int(np.prod(shape))
    b1, b2 = _threefry2x32(k1, k2, np.zeros(2, np.uint64), np.arange(2, dtype=np.uint64))
    keys = np.stack([b1, b2], axis=1)
    cnt_hi = np.zeros(size, np.uint64)
    cnt_lo = np.arange(size, dtype=np.uint64)
    h1, h2 = _threefry2x32(keys[0][0], keys[0][1], cnt_hi, cnt_lo)
    higher = (h1 ^ h2).reshape(shape)
    h1, h2 = _threefry2x32(keys[1][0], keys[1][1], cnt_hi, cnt_lo)
    lower = (h1 ^ h2).reshape(shape)
    span = np.uint64(maxval - minval)
    mult = np.uint64(2**16) % span
    mult = (mult * mult) % span
    off = ((higher.astype(np.uint64) % span) * mult + lower.astype(np.uint64) % span) & _U32
    off = off % span
    return (np.int64(minval) + off.astype(np.int64)).astype(np.int32)


def _pair_indices(n):
    if n not in _IDX_CACHE:
        idx = _randint_np(42, (2, MAX_PAIRS), 0, n)
        order = np.argsort(idx[0], kind="stable")
        ii_s = idx[0][order]
        jj_s = idx[1][order]
        rows = n // NUM_WORKERS
        counts = np.bincount(ii_s // rows, minlength=NUM_WORKERS)
        quantum = NCHUNK * LANES
        per_worker = int(-(-counts.max() // quantum) * quantum)
        li = np.zeros((NUM_WORKERS, per_worker), np.int32)
        jj = np.zeros((NUM_WORKERS, per_worker), np.int32)
        starts = np.concatenate([[0], np.cumsum(counts)])
        for w in range(NUM_WORKERS):
            c = counts[w]
            li[w, :c] = ii_s[starts[w]:starts[w + 1]] - w * rows
            jj[w, :c] = jj_s[starts[w]:starts[w + 1]]
            jj[w, c:] = w * rows
        li_packed = li | ((jj & 3) << 20)
        jj4 = jj >> 2
        _IDX_CACHE[n] = (li_packed.reshape(-1), jj4.reshape(-1), per_worker)
    return _IDX_CACHE[n]


def _make_z_kernel(n):
    mesh = plsc.VectorSubcoreMesh(
        core_axis_name="c", subcore_axis_name="s", num_cores=NUM_CORES
    )
    rows = n // NUM_WORKERS
    CH = 4096
    NCH = rows // CH

    @functools.partial(
        pl.kernel,
        mesh=mesh,
        out_type=jax.ShapeDtypeStruct((n // 4, 8), jnp.float32),
        compiler_params=pltpu.CompilerParams(
            needs_layout_passes=False, use_tc_tiling_on_sc=False
        ),
        scratch_types=[
            pltpu.VMEM((CH,), jnp.float32),
            pltpu.VMEM((CH,), jnp.float32),
            pltpu.VMEM((CH // 4, 8), jnp.float32),
        ],
    )
    def z_build_kernel(o_hbm, y_hbm, z_out, ob, yb, zv):
        wid = lax.axis_index("s") * NUM_CORES + lax.axis_index("c")
        rbase = wid * rows
        iota = lax.iota(jnp.int32, LANES)
        row4 = lax.shift_right_logical(iota, 2)
        colo = 2 * (iota & 3)
        coly = colo + 1

        for t in range(NCH):
            s = rbase + t * CH
            pltpu.sync_copy(o_hbm.at[pl.ds(s, CH)], ob)
            pltpu.sync_copy(y_hbm.at[pl.ds(s, CH)], yb)

            def body(k, carry):
                sl = pl.ds(k * LANES, LANES)
                rowi = k * 4 + row4
                plsc.store_scatter(zv, [rowi, colo], ob[sl])
                plsc.store_scatter(zv, [rowi, coly], yb[sl])
                return carry

            lax.fori_loop(0, CH // LANES, body, 0, unroll=4)
            pltpu.sync_copy(zv, z_out.at[pl.ds(s // 4, CH // 4)])

    return z_build_kernel


def _make_sc_kernel(n, per_worker):
    mesh = plsc.VectorSubcoreMesh(
        core_axis_name="c", subcore_axis_name="s", num_cores=NUM_CORES
    )
    B = per_worker
    C = B // NCHUNK
    CV = C // LANES
    rows = n // NUM_WORKERS

    @functools.partial(
        pl.kernel,
        mesh=mesh,
        out_type=jax.ShapeDtypeStruct((2 * NUM_WORKERS, LANES), jnp.float32),
        compiler_params=pltpu.CompilerParams(
            needs_layout_passes=False, use_tc_tiling_on_sc=False
        ),
        scratch_types=[
            pltpu.VMEM((rows,), jnp.float32),
            pltpu.VMEM((rows,), jnp.float32),
            pltpu.VMEM((B,), jnp.int32),
            pltpu.VMEM((B,), jnp.int32),
            pltpu.VMEM((C, 8), jnp.float32),
            pltpu.VMEM((C, 8), jnp.float32),
            pltpu.VMEM((LANES,), jnp.float32),
            pltpu.VMEM((LANES,), jnp.float32),
            pltpu.SemaphoreType.DMA,
            pltpu.SemaphoreType.DMA,
            pltpu.SemaphoreType.DMA,
        ],
    )
    def ranking_loss_kernel(
        o_hbm, y_hbm, z_hbm, li_hbm, jj_hbm, out_hbm,
        or_v, yr_v, li_v, jj_v, zj0, zj1, nv, dv, semg0, semg1, seml,
    ):
        wid = lax.axis_index("s") * NUM_CORES + lax.axis_index("c")
        base = wid * B
        rbase = wid * rows
        zbufs = (zj0, zj1)
        sems = (semg0, semg1)
        pltpu.sync_copy(jj_hbm.at[pl.ds(base, B)], jj_v)

        def fire(c):
            idx = jj_v.at[pl.ds(c * C, C)]
            return pltpu.async_copy(z_hbm.at[idx], zbufs[c % 2], sems[c % 2])

        handles = [fire(0), fire(1)]
        c2 = pltpu.async_copy(o_hbm.at[pl.ds(rbase, rows)], or_v, seml)
        c3 = pltpu.async_copy(y_hbm.at[pl.ds(rbase, rows)], yr_v, seml)
        c4 = pltpu.async_copy(li_hbm.at[pl.ds(base, B)], li_v, seml)
        c2.wait()
        c3.wait()
        c4.wait()

        iota = lax.iota(jnp.int32, LANES)
        zeros = jnp.zeros((LANES,), jnp.float32)
        num = zeros
        den = zeros
        for c in range(NCHUNK):
            handles[c].wait()
            zb = zbufs[c % 2]

            def body(k, carry, c=c, zb=zb):
                num, den = carry
                sl = pl.ds(c * C + k * LANES, LANES)
                raw = li_v[sl]
                liv = raw & 0xFFFF
                colo = lax.shift_right_logical(raw, 19)
                prow = k * LANES + iota
                o_i = plsc.load_gather(or_v, [liv])
                y_i = plsc.load_gather(yr_v, [liv])
                o_j = plsc.load_gather(zb, [prow, colo])
                y_j = plsc.load_gather(zb, [prow, colo + 1])
                d_o = o_i - o_j
                d_y = y_i - y_j
                t = jnp.sign(d_y)
                num = num + jnp.maximum(0.0, -t * d_o)
                den = den + jnp.where(d_y != 0.0, 1.0, 0.0)
                return num, den

            num, den = lax.fori_loop(0, CV, body, (num, den), unroll=4)
            if c + 2 < NCHUNK:
                handles.append(fire(c + 2))
        nv[...] = num
        dv[...] = den
        pltpu.sync_copy(nv, out_hbm.at[wid])
        pltpu.sync_copy(dv, out_hbm.at[NUM_WORKERS + wid])

    return ranking_loss_kernel


_KERNEL_CACHE = {}


def kernel(outputs, y):
    o = outputs.reshape(-1)
    yy = y.reshape(-1)
    n = o.shape[0]
    li_np, jj_np, per_worker = _pair_indices(n)
    if n not in _KERNEL_CACHE:
        _KERNEL_CACHE[n] = (_make_z_kernel(n), _make_sc_kernel(n, per_worker))
    z_kernel, loss_kernel = _KERNEL_CACHE[n]
    li = jnp.asarray(li_np)
    jj = jnp.asarray(jj_np)
    z = z_kernel(o, yy)
    partials = loss_kernel(o, yy, z, li, jj)
    num = jnp.sum(partials[:NUM_WORKERS])
    den = jnp.sum(partials[NUM_WORKERS:])
    return num / den


# speedup vs baseline: 1.1925x; 1.1925x over previous
import functools

import jax
import jax.numpy as jnp
import numpy as np
from jax import lax
from jax.experimental import pallas as pl
from jax.experimental.pallas import tpu as pltpu
from jax.experimental.pallas import tpu_sc as plsc

MAX_PAIRS = 500000
NUM_CORES = 2
NUM_SUBCORES = 16
NUM_WORKERS = NUM_CORES * NUM_SUBCORES
LANES = 16
NCHUNK = 8

PAIRS_PER_WORKER = -(-MAX_PAIRS // (NUM_WORKERS * LANES)) * LANES
PADDED_PAIRS = PAIRS_PER_WORKER * NUM_WORKERS
VECS_PER_WORKER = PAIRS_PER_WORKER // LANES

_IDX_CACHE = {}

_U32 = np.uint64(0xFFFFFFFF)


def _threefry2x32(k1, k2, x0, x1):
    k1 = np.uint64(k1)
    k2 = np.uint64(k2)
    x0 = np.asarray(x0, np.uint64)
    x1 = np.asarray(x1, np.uint64)
    ks = [k1 & _U32, k2 & _U32, (k1 ^ k2 ^ np.uint64(0x1BD11BDA)) & _U32]
    rot = ([13, 15, 26, 6], [17, 29, 16, 24])
    x0 = (x0 + ks[0]) & _U32
    x1 = (x1 + ks[1]) & _U32
    for i in range(5):
        for r in rot[i % 2]:
            x0 = (x0 + x1) & _U32
            x1 = ((x1 << np.uint64(r)) | (x1 >> np.uint64(32 - r))) & _U32
            x1 = x0 ^ x1
        x0 = (x0 + ks[(i + 1) % 3]) & _U32
        x1 = (x1 + ks[(i + 2) % 3] + np.uint64(i + 1)) & _U32
    return x0.astype(np.uint32), x1.astype(np.uint32)


def _randint_np(seed, shape, minval, maxval):
    k1 = np.uint32((seed >> 32) & 0xFFFFFFFF)
    k2 = np.uint32(seed & 0xFFFFFFFF)
    size = int(np.prod(shape))
    b1, b2 = _threefry2x32(k1, k2, np.zeros(2, np.uint64), np.arange(2, dtype=np.uint64))
    keys = np.stack([b1, b2], axis=1)
    cnt_hi = np.zeros(size, np.uint64)
    cnt_lo = np.arange(size, dtype=np.uint64)
    h1, h2 = _threefry2x32(keys[0][0], keys[0][1], cnt_hi, cnt_lo)
    higher = (h1 ^ h2).reshape(shape)
    h1, h2 = _threefry2x32(keys[1][0], keys[1][1], cnt_hi, cnt_lo)
    lower = (h1 ^ h2).reshape(shape)
    span = np.uint64(maxval - minval)
    mult = np.uint64(2**16) % span
    mult = (mult * mult) % span
    off = ((higher.astype(np.uint64) % span) * mult + lower.astype(np.uint64) % span) & _U32
    off = off % span
    return (np.int64(minval) + off.astype(np.int64)).astype(np.int32)


def _pair_indices(n):
    if n not in _IDX_CACHE:
        idx = _randint_np(42, (2, MAX_PAIRS), 0, n)
        order = np.argsort(idx[0], kind="stable")
        ii_s = idx[0][order]
        jj_s = idx[1][order]
        rows = n // NUM_WORKERS
        counts = np.bincount(ii_s // rows, minlength=NUM_WORKERS)
        quantum = NCHUNK * LANES
        per_worker = int(-(-counts.max() // quantum) * quantum)
        li = np.zeros((NUM_WORKERS, per_worker), np.int32)
        jj = np.zeros((NUM_WORKERS, per_worker), np.int32)
        starts = np.concatenate([[0], np.cumsum(counts)])
        for w in range(NUM_WORKERS):
            c = counts[w]
            li[w, :c] = ii_s[starts[w]:starts[w + 1]] - w * rows
            jj[w, :c] = jj_s[starts[w]:starts[w + 1]]
            jj[w, c:] = w * rows
        li_packed = li | ((jj & 3) << 20)
        jj4 = jj >> 2
        _IDX_CACHE[n] = (li_packed.reshape(-1), jj4.reshape(-1), per_worker)
    return _IDX_CACHE[n]


def _make_z_kernel(n):
    mesh = plsc.VectorSubcoreMesh(
        core_axis_name="c", subcore_axis_name="s", num_cores=NUM_CORES
    )
    rows = n // NUM_WORKERS
    CH = 8192
    NCH = rows // CH

    @functools.partial(
        pl.kernel,
        mesh=mesh,
        out_type=jax.ShapeDtypeStruct((n // 4, 8), jnp.float32),
        compiler_params=pltpu.CompilerParams(
            needs_layout_passes=False, use_tc_tiling_on_sc=False
        ),
        scratch_types=[
            pltpu.VMEM((CH,), jnp.float32),
            pltpu.VMEM((CH,), jnp.float32),
            pltpu.VMEM((CH,), jnp.float32),
            pltpu.VMEM((CH,), jnp.float32),
            pltpu.VMEM((CH // 4, 8), jnp.float32),
            pltpu.VMEM((CH // 4, 8), jnp.float32),
            pltpu.SemaphoreType.DMA,
            pltpu.SemaphoreType.DMA,
            pltpu.SemaphoreType.DMA,
            pltpu.SemaphoreType.DMA,
        ],
    )
    def z_build_kernel(
        o_hbm, y_hbm, z_out, ob0, yb0, ob1, yb1, zv0, zv1,
        semi0, semi1, semo0, semo1,
    ):
        wid = lax.axis_index("s") * NUM_CORES + lax.axis_index("c")
        rbase = wid * rows
        iota = lax.iota(jnp.int32, LANES)
        row4 = lax.shift_right_logical(iota, 2)
        colo = 2 * (iota & 3)
        coly = colo + 1
        obufs = ((ob0, yb0), (ob1, yb1))
        zvs = (zv0, zv1)
        semi = (semi0, semi1)
        semo = (semo0, semo1)

        def fire_in(t):
            p = t % 2
            s = rbase + t * CH
            ob, yb = obufs[p]
            h1 = pltpu.async_copy(o_hbm.at[pl.ds(s, CH)], ob, semi[p])
            h2 = pltpu.async_copy(y_hbm.at[pl.ds(s, CH)], yb, semi[p])
            return (h1, h2)

        hin = [fire_in(0), fire_in(1)]
        hout = [None, None]
        for t in range(NCH):
            p = t % 2
            for h in hin[t]:
                h.wait()
            if hout[p] is not None:
                hout[p].wait()
            ob, yb = obufs[p]
            zv = zvs[p]

            def body(k, carry, ob=ob, yb=yb, zv=zv):
                sl = pl.ds(k * LANES, LANES)
                rowi = k * 4 + row4
                plsc.store_scatter(zv, [rowi, colo], ob[sl])
                plsc.store_scatter(zv, [rowi, coly], yb[sl])
                return carry

            lax.fori_loop(0, CH // LANES, body, 0, unroll=4)
            s = rbase + t * CH
            hout[p] = pltpu.async_copy(
                zv, z_out.at[pl.ds(s // 4, CH // 4)], semo[p]
            )
            if t + 2 < NCH:
                hin.append(fire_in(t + 2))
        hout[0].wait()
        hout[1].wait()

    return z_build_kernel


def _make_sc_kernel(n, per_worker):
    mesh = plsc.VectorSubcoreMesh(
        core_axis_name="c", subcore_axis_name="s", num_cores=NUM_CORES
    )
    B = per_worker
    C = B // NCHUNK
    CV = C // LANES
    rows = n // NUM_WORKERS

    @functools.partial(
        pl.kernel,
        mesh=mesh,
        out_type=jax.ShapeDtypeStruct((2 * NUM_WORKERS, LANES), jnp.float32),
        compiler_params=pltpu.CompilerParams(
            needs_layout_passes=False, use_tc_tiling_on_sc=False
        ),
        scratch_types=[
            pltpu.VMEM((rows,), jnp.float32),
            pltpu.VMEM((rows,), jnp.float32),
            pltpu.VMEM((B,), jnp.int32),
            pltpu.VMEM((B,), jnp.int32),
            pltpu.VMEM((C, 8), jnp.float32),
            pltpu.VMEM((C, 8), jnp.float32),
            pltpu.VMEM((LANES,), jnp.float32),
            pltpu.VMEM((LANES,), jnp.float32),
            pltpu.SemaphoreType.DMA,
            pltpu.SemaphoreType.DMA,
            pltpu.SemaphoreType.DMA,
        ],
    )
    def ranking_loss_kernel(
        o_hbm, y_hbm, z_hbm, li_hbm, jj_hbm, out_hbm,
        or_v, yr_v, li_v, jj_v, zj0, zj1, nv, dv, semg0, semg1, seml,
    ):
        wid = lax.axis_index("s") * NUM_CORES + lax.axis_index("c")
        base = wid * B
        rbase = wid * rows
        zbufs = (zj0, zj1)
        sems = (semg0, semg1)
        pltpu.sync_copy(jj_hbm.at[pl.ds(base, B)], jj_v)

        def fire(c):
            idx = jj_v.at[pl.ds(c * C, C)]
            return pltpu.async_copy(z_hbm.at[idx], zbufs[c % 2], sems[c % 2])

        handles = [fire(0), fire(1)]
        c2 = pltpu.async_copy(o_hbm.at[pl.ds(rbase, rows)], or_v, seml)
        c3 = pltpu.async_copy(y_hbm.at[pl.ds(rbase, rows)], yr_v, seml)
        c4 = pltpu.async_copy(li_hbm.at[pl.ds(base, B)], li_v, seml)
        c2.wait()
        c3.wait()
        c4.wait()

        iota = lax.iota(jnp.int32, LANES)
        zeros = jnp.zeros((LANES,), jnp.float32)
        num = zeros
        den = zeros
        for c in range(NCHUNK):
            handles[c].wait()
            zb = zbufs[c % 2]

            def body(k, carry, c=c, zb=zb):
                num, den = carry
                sl = pl.ds(c * C + k * LANES, LANES)
                raw = li_v[sl]
                liv = raw & 0xFFFF
                colo = lax.shift_right_logical(raw, 19)
                prow = k * LANES + iota
                o_i = plsc.load_gather(or_v, [liv])
                y_i = plsc.load_gather(yr_v, [liv])
                o_j = plsc.load_gather(zb, [prow, colo])
                y_j = plsc.load_gather(zb, [prow, colo + 1])
                d_o = o_i - o_j
                d_y = y_i - y_j
                t = jnp.sign(d_y)
                num = num + jnp.maximum(0.0, -t * d_o)
                den = den + jnp.where(d_y != 0.0, 1.0, 0.0)
                return num, den

            num, den = lax.fori_loop(0, CV, body, (num, den), unroll=4)
            if c + 2 < NCHUNK:
                handles.append(fire(c + 2))
        nv[...] = num
        dv[...] = den
        pltpu.sync_copy(nv, out_hbm.at[wid])
        pltpu.sync_copy(dv, out_hbm.at[NUM_WORKERS + wid])

    return ranking_loss_kernel


_KERNEL_CACHE = {}


def kernel(outputs, y):
    o = outputs.reshape(-1)
    yy = y.reshape(-1)
    n = o.shape[0]
    li_np, jj_np, per_worker = _pair_indices(n)
    if n not in _KERNEL_CACHE:
        _KERNEL_CACHE[n] = (_make_z_kernel(n), _make_sc_kernel(n, per_worker))
    z_kernel, loss_kernel = _KERNEL_CACHE[n]
    li = jnp.asarray(li_np)
    jj = jnp.asarray(jj_np)
    z = z_kernel(o, yy)
    partials = loss_kernel(o, yy, z, li, jj)
    num = jnp.sum(partials[:NUM_WORKERS])
    den = jnp.sum(partials[NUM_WORKERS:])
    return num / den


# speedup vs baseline: 1.2245x; 1.0269x over previous
import functools

import jax
import jax.numpy as jnp
import numpy as np
from jax import lax
from jax.experimental import pallas as pl
from jax.experimental.pallas import tpu as pltpu
from jax.experimental.pallas import tpu_sc as plsc

MAX_PAIRS = 500000
NUM_CORES = 2
NUM_SUBCORES = 16
NUM_WORKERS = NUM_CORES * NUM_SUBCORES
LANES = 16
NCHUNK = 8

PAIRS_PER_WORKER = -(-MAX_PAIRS // (NUM_WORKERS * LANES)) * LANES
PADDED_PAIRS = PAIRS_PER_WORKER * NUM_WORKERS
VECS_PER_WORKER = PAIRS_PER_WORKER // LANES

_IDX_CACHE = {}

_U32 = np.uint64(0xFFFFFFFF)


def _threefry2x32(k1, k2, x0, x1):
    k1 = np.uint64(k1)
    k2 = np.uint64(k2)
    x0 = np.asarray(x0, np.uint64)
    x1 = np.asarray(x1, np.uint64)
    ks = [k1 & _U32, k2 & _U32, (k1 ^ k2 ^ np.uint64(0x1BD11BDA)) & _U32]
    rot = ([13, 15, 26, 6], [17, 29, 16, 24])
    x0 = (x0 + ks[0]) & _U32
    x1 = (x1 + ks[1]) & _U32
    for i in range(5):
        for r in rot[i % 2]:
            x0 = (x0 + x1) & _U32
            x1 = ((x1 << np.uint64(r)) | (x1 >> np.uint64(32 - r))) & _U32
            x1 = x0 ^ x1
        x0 = (x0 + ks[(i + 1) % 3]) & _U32
        x1 = (x1 + ks[(i + 2) % 3] + np.uint64(i + 1)) & _U32
    return x0.astype(np.uint32), x1.astype(np.uint32)


def _randint_np(seed, shape, minval, maxval):
    k1 = np.uint32((seed >> 32) & 0xFFFFFFFF)
    k2 = np.uint32(seed & 0xFFFFFFFF)
    size = int(np.prod(shape))
    b1, b2 = _threefry2x32(k1, k2, np.zeros(2, np.uint64), np.arange(2, dtype=np.uint64))
    keys = np.stack([b1, b2], axis=1)
    cnt_hi = np.zeros(size, np.uint64)
    cnt_lo = np.arange(size, dtype=np.uint64)
    h1, h2 = _threefry2x32(keys[0][0], keys[0][1], cnt_hi, cnt_lo)
    higher = (h1 ^ h2).reshape(shape)
    h1, h2 = _threefry2x32(keys[1][0], keys[1][1], cnt_hi, cnt_lo)
    lower = (h1 ^ h2).reshape(shape)
    span = np.uint64(maxval - minval)
    mult = np.uint64(2**16) % span
    mult = (mult * mult) % span
    off = ((higher.astype(np.uint64) % span) * mult + lower.astype(np.uint64) % span) & _U32
    off = off % span
    return (np.int64(minval) + off.astype(np.int64)).astype(np.int32)


def _pair_indices(n):
    if n not in _IDX_CACHE:
        idx = _randint_np(42, (2, MAX_PAIRS), 0, n)
        order = np.argsort(idx[0], kind="stable")
        ii_s = idx[0][order]
        jj_s = idx[1][order]
        rows = n // NUM_WORKERS
        counts = np.bincount(ii_s // rows, minlength=NUM_WORKERS)
        quantum = NCHUNK * LANES
        per_worker = int(-(-counts.max() // quantum) * quantum)
        li = np.zeros((NUM_WORKERS, per_worker), np.int32)
        jj = np.zeros((NUM_WORKERS, per_worker), np.int32)
        starts = np.concatenate([[0], np.cumsum(counts)])
        for w in range(NUM_WORKERS):
            c = counts[w]
            li[w, :c] = ii_s[starts[w]:starts[w + 1]] - w * rows
            jj[w, :c] = jj_s[starts[w]:starts[w + 1]]
            jj[w, c:] = w * rows
        _IDX_CACHE[n] = (li.reshape(-1), jj.reshape(-1), per_worker)
    return _IDX_CACHE[n]


def _make_sc_kernel(n, per_worker):
    mesh = plsc.VectorSubcoreMesh(
        core_axis_name="c", subcore_axis_name="s", num_cores=NUM_CORES
    )
    B = per_worker
    C = B // NCHUNK
    CV = C // LANES
    rows = n // NUM_WORKERS

    @functools.partial(
        pl.kernel,
        mesh=mesh,
        out_type=jax.ShapeDtypeStruct((2 * NUM_WORKERS, LANES), jnp.float32),
        compiler_params=pltpu.CompilerParams(needs_layout_passes=False),
        scratch_types=[
            pltpu.VMEM((rows,), jnp.float32),
            pltpu.VMEM((rows,), jnp.float32),
            pltpu.VMEM((B,), jnp.int32),
            pltpu.VMEM((B,), jnp.int32),
            pltpu.VMEM((C,), jnp.float32),
            pltpu.VMEM((C,), jnp.float32),
            pltpu.VMEM((C,), jnp.float32),
            pltpu.VMEM((C,), jnp.float32),
            pltpu.VMEM((LANES,), jnp.float32),
            pltpu.VMEM((LANES,), jnp.float32),
            pltpu.SemaphoreType.DMA,
            pltpu.SemaphoreType.DMA,
            pltpu.SemaphoreType.DMA,
        ],
    )
    def ranking_loss_kernel(
        o_hbm, y_hbm, li_hbm, jj_hbm, out_hbm,
        or_v, yr_v, li_v, jj_v, oj0, yj0, oj1, yj1, nv, dv, semg0, semg1, seml,
    ):
        wid = lax.axis_index("s") * NUM_CORES + lax.axis_index("c")
        base = wid * B
        rbase = wid * rows
        zbufs = ((oj0, yj0), (oj1, yj1))
        sems = (semg0, semg1)
        pltpu.sync_copy(jj_hbm.at[pl.ds(base, B)], jj_v)

        def fire(c):
            idx = jj_v.at[pl.ds(c * C, C)]
            ob, yb = zbufs[c % 2]
            h1 = pltpu.async_copy(o_hbm.at[idx], ob, sems[c % 2])
            h2 = pltpu.async_copy(y_hbm.at[idx], yb, sems[c % 2])
            return (h1, h2)

        handles = [fire(0), fire(1)]
        c2 = pltpu.async_copy(o_hbm.at[pl.ds(rbase, rows)], or_v, seml)
        c3 = pltpu.async_copy(y_hbm.at[pl.ds(rbase, rows)], yr_v, seml)
        c4 = pltpu.async_copy(li_hbm.at[pl.ds(base, B)], li_v, seml)
        c2.wait()
        c3.wait()
        c4.wait()

        zeros = jnp.zeros((LANES,), jnp.float32)
        num = zeros
        den = zeros
        for c in range(NCHUNK):
            h1, h2 = handles[c]
            h1.wait()
            h2.wait()
            ob, yb = zbufs[c % 2]

            def body(k, carry, c=c, ob=ob, yb=yb):
                num, den = carry
                sl = pl.ds(c * C + k * LANES, LANES)
                cl = pl.ds(k * LANES, LANES)
                liv = li_v[sl]
                o_i = plsc.load_gather(or_v, [liv])
                y_i = plsc.load_gather(yr_v, [liv])
                d_o = o_i - ob[cl]
                d_y = y_i - yb[cl]
                t = jnp.sign(d_y)
                num = num + jnp.maximum(0.0, -t * d_o)
                den = den + jnp.where(d_y != 0.0, 1.0, 0.0)
                return num, den

            num, den = lax.fori_loop(0, CV, body, (num, den), unroll=4)
            if c + 2 < NCHUNK:
                handles.append(fire(c + 2))
        nv[...] = num
        dv[...] = den
        pltpu.sync_copy(nv, out_hbm.at[wid])
        pltpu.sync_copy(dv, out_hbm.at[NUM_WORKERS + wid])

    return ranking_loss_kernel


_KERNEL_CACHE = {}


def kernel(outputs, y):
    o = outputs.reshape(-1)
    yy = y.reshape(-1)
    n = o.shape[0]
    li_np, jj_np, per_worker = _pair_indices(n)
    if n not in _KERNEL_CACHE:
        _KERNEL_CACHE[n] = _make_sc_kernel(n, per_worker)
    li = jnp.asarray(li_np)
    jj = jnp.asarray(jj_np)
    partials = _KERNEL_CACHE[n](o, yy, li, jj)
    num = jnp.sum(partials[:NUM_WORKERS])
    den = jnp.sum(partials[NUM_WORKERS:])
    return num / den
